# initial kernel scaffold (unmeasured)
import jax
import jax.numpy as jnp
from jax import lax
from jax.experimental import pallas as pl
from jax.experimental.pallas import tpu as pltpu

N_DEV = 4


def _body(x_ref, w_ref, out_ref, recv_ref, rs_send, rs_recv, ag_send, ag_recv):
    m, n = out_ref.shape
    chunk = m // N_DEV
    my_pos = lax.axis_index("i")
    left = lax.rem(my_pos + N_DEV - 1, N_DEV)
    right = lax.rem(my_pos + 1, N_DEV)

    barrier = pltpu.get_barrier_semaphore()
    for nbr in (left, right):
        pl.semaphore_signal(
            barrier, inc=1, device_id=(nbr,), device_id_type=pl.DeviceIdType.MESH
        )
    pl.semaphore_wait(barrier, 2)

    for c in range(N_DEV):
        out_ref[pl.ds(c * chunk, chunk), :] = jnp.dot(
            x_ref[pl.ds(c * chunk, chunk), :],
            w_ref[...],
            preferred_element_type=jnp.float32,
        )

    for s in range(N_DEV - 1):
        send_c = lax.rem(my_pos - s + N_DEV, N_DEV)
        acc_c = lax.rem(my_pos - s - 1 + N_DEV, N_DEV)
        rdma = pltpu.make_async_remote_copy(
            src_ref=out_ref.at[pl.ds(send_c * chunk, chunk), :],
            dst_ref=recv_ref.at[s % 2],
            send_sem=rs_send.at[s],
            recv_sem=rs_recv.at[s],
            device_id=(right,),
            device_id_type=pl.DeviceIdType.MESH,
        )
        rdma.start()
        rdma.wait()
        out_ref[pl.ds(acc_c * chunk, chunk), :] += recv_ref[s % 2]

    for s in range(N_DEV - 1):
        c = lax.rem(my_pos + 1 - s + N_DEV, N_DEV)
        rdma = pltpu.make_async_remote_copy(
            src_ref=out_ref.at[pl.ds(c * chunk, chunk), :],
            dst_ref=out_ref.at[pl.ds(c * chunk, chunk), :],
            send_sem=ag_send.at[s],
            recv_sem=ag_recv.at[s],
            device_id=(right,),
            device_id_type=pl.DeviceIdType.MESH,
        )
        rdma.start()
        rdma.wait()

    amax = jnp.float32(0.0)
    for c in range(N_DEV):
        amax = jnp.maximum(
            amax, jnp.max(jnp.abs(out_ref[pl.ds(c * chunk, chunk), :]))
        )
    scale = amax / 448.0
    for c in range(N_DEV):
        y = out_ref[pl.ds(c * chunk, chunk), :]
        q = jnp.clip(y / scale, -448.0, 448.0).astype(jnp.float8_e4m3fn)
        out_ref[pl.ds(c * chunk, chunk), :] = q.astype(jnp.float32) * scale


def kernel(x, w_mat):
    x = x.astype(jnp.bfloat16)
    w = w_mat.astype(jnp.bfloat16)
    m, _ = x.shape
    n = w.shape[1]
    chunk = m // N_DEV
    return pl.pallas_call(
        _body,
        out_shape=jax.ShapeDtypeStruct((m, n), jnp.float32),
        in_specs=[
            pl.BlockSpec(memory_space=pltpu.VMEM),
            pl.BlockSpec(memory_space=pltpu.VMEM),
        ],
        out_specs=pl.BlockSpec(memory_space=pltpu.VMEM),
        scratch_shapes=[
            pltpu.VMEM((2, chunk, n), jnp.float32),
            pltpu.SemaphoreType.DMA((N_DEV - 1,)),
            pltpu.SemaphoreType.DMA((N_DEV - 1,)),
            pltpu.SemaphoreType.DMA((N_DEV - 1,)),
            pltpu.SemaphoreType.DMA((N_DEV - 1,)),
        ],
        compiler_params=pltpu.CompilerParams(collective_id=0),
    )(x, w)


# baseline (device time: 648433 ns/iter reference)
import jax
import jax.numpy as jnp
from jax import lax
from jax.experimental import pallas as pl
from jax.experimental.pallas import tpu as pltpu

N_DEV = 4
SUB = 4


def _body(x_ref, w_ref, out_ref, recv_ref, rs_send, rs_recv, ag_send, ag_recv):
    m, n = out_ref.shape
    chunk = m // N_DEV
    sub = chunk // SUB
    my_pos = lax.axis_index("i")
    left = lax.rem(my_pos + N_DEV - 1, N_DEV)
    right = lax.rem(my_pos + 1, N_DEV)

    barrier = pltpu.get_barrier_semaphore()
    for nbr in (left, right):
        pl.semaphore_signal(
            barrier, inc=1, device_id=(nbr,), device_id_type=pl.DeviceIdType.MESH
        )
    pl.semaphore_wait(barrier, 2)

    for c in range(N_DEV):
        out_ref[pl.ds(c * chunk, chunk), :] = jnp.dot(
            x_ref[pl.ds(c * chunk, chunk), :],
            w_ref[...],
            preferred_element_type=jnp.float32,
        )

    for s in range(N_DEV - 1):
        send_c = lax.rem(my_pos - s + N_DEV, N_DEV)
        acc_c = lax.rem(my_pos - s - 1 + N_DEV, N_DEV)
        for j in range(SUB):
            slot = (s * SUB + j) % 2
            rdma = pltpu.make_async_remote_copy(
                src_ref=out_ref.at[pl.ds(send_c * chunk + j * sub, sub), :],
                dst_ref=recv_ref.at[slot],
                send_sem=rs_send.at[s, j],
                recv_sem=rs_recv.at[s, j],
                device_id=(right,),
                device_id_type=pl.DeviceIdType.MESH,
            )
            rdma.start()
            rdma.wait()
            out_ref[pl.ds(acc_c * chunk + j * sub, sub), :] += recv_ref[slot]

    for s in range(N_DEV - 1):
        c = lax.rem(my_pos + 1 - s + N_DEV, N_DEV)
        rdma = pltpu.make_async_remote_copy(
            src_ref=out_ref.at[pl.ds(c * chunk, chunk), :],
            dst_ref=out_ref.at[pl.ds(c * chunk, chunk), :],
            send_sem=ag_send.at[s],
            recv_sem=ag_recv.at[s],
            device_id=(right,),
            device_id_type=pl.DeviceIdType.MESH,
        )
        rdma.start()
        rdma.wait()

    amax = jnp.float32(0.0)
    for c in range(N_DEV * SUB):
        amax = jnp.maximum(
            amax, jnp.max(jnp.abs(out_ref[pl.ds(c * sub, sub), :]))
        )
    scale = amax / 448.0
    for c in range(N_DEV * SUB):
        y = out_ref[pl.ds(c * sub, sub), :]
        q = jnp.clip(y / scale, -448.0, 448.0).astype(jnp.float8_e4m3fn)
        out_ref[pl.ds(c * sub, sub), :] = q.astype(jnp.float32) * scale


def kernel(x, w_mat):
    x = x.astype(jnp.bfloat16)
    w = w_mat.astype(jnp.bfloat16)
    m, _ = x.shape
    n = w.shape[1]
    chunk = m // N_DEV
    return pl.pallas_call(
        _body,
        out_shape=jax.ShapeDtypeStruct((m, n), jnp.float32),
        in_specs=[
            pl.BlockSpec(memory_space=pltpu.VMEM),
            pl.BlockSpec(memory_space=pltpu.VMEM),
        ],
        out_specs=pl.BlockSpec(memory_space=pltpu.VMEM),
        scratch_shapes=[
            pltpu.VMEM((2, chunk // SUB, n), jnp.float32),
            pltpu.SemaphoreType.DMA((N_DEV - 1, SUB)),
            pltpu.SemaphoreType.DMA((N_DEV - 1, SUB)),
            pltpu.SemaphoreType.DMA((N_DEV - 1,)),
            pltpu.SemaphoreType.DMA((N_DEV - 1,)),
        ],
        compiler_params=pltpu.CompilerParams(
            collective_id=0, vmem_limit_bytes=100 * 1024 * 1024
        ),
    )(x, w)


# device time: 281131 ns/iter; 2.3065x vs baseline; 2.3065x over previous
import jax
import jax.numpy as jnp
from jax import lax
from jax.experimental import pallas as pl
from jax.experimental.pallas import tpu as pltpu

N_DEV = 4
SUB = 4


def _body(
    x_ref,
    w_ref,
    out_ref,
    sendbuf,
    recvbuf,
    agbuf,
    amax_buf,
    rs_send,
    rs_recv,
    ag_send,
    ag_recv,
    am_send,
    am_recv,
    credit,
):
    m, n = out_ref.shape
    chunk = m // N_DEV
    sub = chunk // SUB
    my = lax.axis_index("i")
    left = lax.rem(my + N_DEV - 1, N_DEV)
    right = lax.rem(my + 1, N_DEV)

    barrier = pltpu.get_barrier_semaphore()
    for nbr in (left, right):
        pl.semaphore_signal(
            barrier, inc=1, device_id=(nbr,), device_id_type=pl.DeviceIdType.MESH
        )
    pl.semaphore_wait(barrier, 2)

    def gemm(c):
        out_ref[pl.ds(c * chunk, chunk), :] = jnp.dot(
            x_ref[pl.ds(c * chunk, chunk), :],
            w_ref[...],
            preferred_element_type=jnp.float32,
        )

    gemm(my)
    for s in range(N_DEV - 1):
        c_send = lax.rem(my - s + N_DEV, N_DEV)
        c_next = lax.rem(my - s - 1 + N_DEV, N_DEV)
        if s >= 1:
            pl.semaphore_wait(credit, 1)
        rdmas = []
        for j in range(SUB):
            sendbuf[j] = out_ref[
                pl.ds(c_send * chunk + j * sub, sub), :
            ].astype(jnp.bfloat16)
            r = pltpu.make_async_remote_copy(
                src_ref=sendbuf.at[j],
                dst_ref=recvbuf.at[j],
                send_sem=rs_send.at[s, j],
                recv_sem=rs_recv.at[s, j],
                device_id=(right,),
                device_id_type=pl.DeviceIdType.MESH,
            )
            r.start()
            rdmas.append(r)
        gemm(c_next)
        for j in range(SUB):
            rdmas[j].wait_recv()
            rows = pl.ds(c_next * chunk + j * sub, sub)
            out_ref[rows, :] += recvbuf[j].astype(jnp.float32)
        if s <= N_DEV - 3:
            pl.semaphore_signal(
                credit, inc=1, device_id=(left,), device_id_type=pl.DeviceIdType.MESH
            )
        for j in range(SUB):
            rdmas[j].wait_send()

    keep = lax.rem(my + 1, N_DEV)
    local_amax = jnp.float32(0.0)
    for h in range(2):
        rows = pl.ds(keep * chunk + h * (chunk // 2), chunk // 2)
        local_amax = jnp.maximum(local_amax, jnp.max(jnp.abs(out_ref[rows, :])))
    amax_buf[pl.ds(my, 1)] = jnp.full((1, 8, 128), local_amax, jnp.float32)
    am_rdmas = []
    for k in range(1, N_DEV):
        peer = lax.rem(my + k, N_DEV)
        r = pltpu.make_async_remote_copy(
            src_ref=amax_buf.at[pl.ds(my, 1)],
            dst_ref=amax_buf.at[pl.ds(my, 1)],
            send_sem=am_send.at[k - 1],
            recv_sem=am_recv.at[k - 1],
            device_id=(peer,),
            device_id_type=pl.DeviceIdType.MESH,
        )
        r.start()
        am_rdmas.append(r)
    for r in am_rdmas:
        r.wait_recv()
    g_amax = jnp.max(amax_buf[...])
    for r in am_rdmas:
        r.wait_send()
    scale = g_amax / 448.0
    inv_scale = 448.0 / g_amax

    def quant_to(slot, c):
        for h in range(2):
            y = out_ref[pl.ds(c * chunk + h * (chunk // 2), chunk // 2), :]
            agbuf[slot, pl.ds(h * (chunk // 2), chunk // 2), :] = jnp.clip(
                y * inv_scale, -448.0, 448.0
            ).astype(jnp.float8_e4m3fn)

    def dequant_from(slot, c):
        for h in range(2):
            rows = pl.ds(h * (chunk // 2), chunk // 2)
            out_ref[pl.ds(c * chunk + h * (chunk // 2), chunk // 2), :] = (
                agbuf[slot, rows, :].astype(jnp.float32) * scale
            )

    quant_to(0, keep)
    for s in range(N_DEV - 1):
        r = pltpu.make_async_remote_copy(
            src_ref=agbuf.at[s % 2],
            dst_ref=agbuf.at[(s + 1) % 2],
            send_sem=ag_send.at[s],
            recv_sem=ag_recv.at[s],
            device_id=(right,),
            device_id_type=pl.DeviceIdType.MESH,
        )
        r.start()
        dequant_from(s % 2, lax.rem(my - s + 1 + N_DEV, N_DEV))
        r.wait()
    dequant_from((N_DEV - 1) % 2, lax.rem(my - 2 + N_DEV, N_DEV))


def kernel(x, w_mat):
    x = x.astype(jnp.bfloat16)
    w = w_mat.astype(jnp.bfloat16)
    m, _ = x.shape
    n = w.shape[1]
    chunk = m // N_DEV
    sub = chunk // SUB
    return pl.pallas_call(
        _body,
        out_shape=jax.ShapeDtypeStruct((m, n), jnp.float32),
        in_specs=[
            pl.BlockSpec(memory_space=pltpu.VMEM),
            pl.BlockSpec(memory_space=pltpu.VMEM),
        ],
        out_specs=pl.BlockSpec(memory_space=pltpu.VMEM),
        scratch_shapes=[
            pltpu.VMEM((SUB, sub, n), jnp.bfloat16),
            pltpu.VMEM((SUB, sub, n), jnp.bfloat16),
            pltpu.VMEM((2, chunk, n), jnp.float8_e4m3fn),
            pltpu.VMEM((N_DEV, 8, 128), jnp.float32),
            pltpu.SemaphoreType.DMA((N_DEV - 1, SUB)),
            pltpu.SemaphoreType.DMA((N_DEV - 1, SUB)),
            pltpu.SemaphoreType.DMA((N_DEV - 1,)),
            pltpu.SemaphoreType.DMA((N_DEV - 1,)),
            pltpu.SemaphoreType.DMA((N_DEV - 1,)),
            pltpu.SemaphoreType.DMA((N_DEV - 1,)),
            pltpu.SemaphoreType.REGULAR,
        ],
        compiler_params=pltpu.CompilerParams(
            collective_id=0, vmem_limit_bytes=100 * 1024 * 1024
        ),
    )(x, w)


# device time: 180562 ns/iter; 3.5912x vs baseline; 1.5570x over previous
import jax
import jax.numpy as jnp
from jax import lax
from jax.experimental import pallas as pl
from jax.experimental.pallas import tpu as pltpu

N_DEV = 4
SUB = 4


def _body(
    x_ref,
    w_ref,
    out_ref,
    sendbuf_r,
    sendbuf_l,
    recvbuf_r,
    recvbuf_l,
    agbuf_r,
    agbuf_l,
    amax_buf,
    rsr_send,
    rsr_recv,
    rsl_send,
    rsl_recv,
    agr_send,
    agr_recv,
    agl_send,
    agl_recv,
    am_send,
    am_recv,
    credit_r,
    credit_l,
):
    m, n = out_ref.shape
    chunk = m // N_DEV
    sub = chunk // SUB
    half = n // 2
    my = lax.axis_index("i")
    left = lax.rem(my + N_DEV - 1, N_DEV)
    right = lax.rem(my + 1, N_DEV)

    barrier = pltpu.get_barrier_semaphore()
    for nbr in (left, right):
        pl.semaphore_signal(
            barrier, inc=1, device_id=(nbr,), device_id_type=pl.DeviceIdType.MESH
        )
    pl.semaphore_wait(barrier, 2)

    def gemm(c):
        out_ref[pl.ds(c * chunk, chunk), :] = jnp.dot(
            x_ref[pl.ds(c * chunk, chunk), :],
            w_ref[...],
            preferred_element_type=jnp.float32,
        )

    gemm(my)
    for s in range(N_DEV - 1):
        cr_s = lax.rem(my - s + N_DEV, N_DEV)
        cl_s = lax.rem(my + s, N_DEV)
        cr_n = lax.rem(my - s - 1 + N_DEV, N_DEV)
        cl_n = lax.rem(my + s + 1, N_DEV)
        if s >= 1:
            pl.semaphore_wait(credit_r, 1)
            pl.semaphore_wait(credit_l, 1)
        rr, rl = [], []
        for j in range(SUB):
            sendbuf_r[j] = out_ref[
                pl.ds(cr_s * chunk + j * sub, sub), pl.ds(0, half)
            ].astype(jnp.bfloat16)
            r = pltpu.make_async_remote_copy(
                src_ref=sendbuf_r.at[j],
                dst_ref=recvbuf_r.at[j],
                send_sem=rsr_send.at[s, j],
                recv_sem=rsr_recv.at[s, j],
                device_id=(right,),
                device_id_type=pl.DeviceIdType.MESH,
            )
            r.start()
            rr.append(r)
        for j in range(SUB):
            sendbuf_l[j] = out_ref[
                pl.ds(cl_s * chunk + j * sub, sub), pl.ds(half, half)
            ].astype(jnp.bfloat16)
            r = pltpu.make_async_remote_copy(
                src_ref=sendbuf_l.at[j],
                dst_ref=recvbuf_l.at[j],
                send_sem=rsl_send.at[s, j],
                recv_sem=rsl_recv.at[s, j],
                device_id=(left,),
                device_id_type=pl.DeviceIdType.MESH,
            )
            r.start()
            rl.append(r)
        if s == 0:
            gemm(cr_n)
            gemm(cl_n)
        elif s == 1:
            gemm(cl_n)
        for j in range(SUB):
            rr[j].wait_recv()
            rows = pl.ds(cr_n * chunk + j * sub, sub)
            out_ref[rows, pl.ds(0, half)] += recvbuf_r[j].astype(jnp.float32)
        for j in range(SUB):
            rl[j].wait_recv()
            rows = pl.ds(cl_n * chunk + j * sub, sub)
            out_ref[rows, pl.ds(half, half)] += recvbuf_l[j].astype(jnp.float32)
        if s <= N_DEV - 3:
            pl.semaphore_signal(
                credit_r, inc=1, device_id=(left,), device_id_type=pl.DeviceIdType.MESH
            )
            pl.semaphore_signal(
                credit_l, inc=1, device_id=(right,), device_id_type=pl.DeviceIdType.MESH
            )
        for j in range(SUB):
            rr[j].wait_send()
            rl[j].wait_send()

    keep_r = lax.rem(my + 1, N_DEV)
    keep_l = lax.rem(my + N_DEV - 1, N_DEV)
    local_amax = jnp.maximum(
        jnp.max(jnp.abs(out_ref[pl.ds(keep_r * chunk, chunk), pl.ds(0, half)])),
        jnp.max(jnp.abs(out_ref[pl.ds(keep_l * chunk, chunk), pl.ds(half, half)])),
    )
    amax_buf[pl.ds(my, 1)] = jnp.full((1, 8, 128), local_amax, jnp.float32)
    am_rdmas = []
    for k in range(1, N_DEV):
        peer = lax.rem(my + k, N_DEV)
        r = pltpu.make_async_remote_copy(
            src_ref=amax_buf.at[pl.ds(my, 1)],
            dst_ref=amax_buf.at[pl.ds(my, 1)],
            send_sem=am_send.at[k - 1],
            recv_sem=am_recv.at[k - 1],
            device_id=(peer,),
            device_id_type=pl.DeviceIdType.MESH,
        )
        r.start()
        am_rdmas.append(r)
    for r in am_rdmas:
        r.wait_recv()
    g_amax = jnp.max(amax_buf[...])
    for r in am_rdmas:
        r.wait_send()
    scale = g_amax / 448.0
    inv_scale = 448.0 / g_amax

    def quant_to(buf, c, c0):
        for h in range(2):
            y = out_ref[pl.ds(c * chunk + h * (chunk // 2), chunk // 2), pl.ds(c0, half)]
            buf[0, pl.ds(h * (chunk // 2), chunk // 2), :] = jnp.clip(
                y * inv_scale, -448.0, 448.0
            ).astype(jnp.float8_e4m3fn)

    def dequant_from(buf, slot, c, c0):
        for h in range(2):
            rows = pl.ds(h * (chunk // 2), chunk // 2)
            out_ref[
                pl.ds(c * chunk + h * (chunk // 2), chunk // 2), pl.ds(c0, half)
            ] = buf[slot, rows, :].astype(jnp.float32) * scale

    quant_to(agbuf_r, keep_r, 0)
    quant_to(agbuf_l, keep_l, half)
    for s in range(N_DEV - 1):
        r1 = pltpu.make_async_remote_copy(
            src_ref=agbuf_r.at[s % 2],
            dst_ref=agbuf_r.at[(s + 1) % 2],
            send_sem=agr_send.at[s],
            recv_sem=agr_recv.at[s],
            device_id=(right,),
            device_id_type=pl.DeviceIdType.MESH,
        )
        r1.start()
        r2 = pltpu.make_async_remote_copy(
            src_ref=agbuf_l.at[s % 2],
            dst_ref=agbuf_l.at[(s + 1) % 2],
            send_sem=agl_send.at[s],
            recv_sem=agl_recv.at[s],
            device_id=(left,),
            device_id_type=pl.DeviceIdType.MESH,
        )
        r2.start()
        if s == 0:
            dequant_from(agbuf_r, 0, keep_r, 0)
            dequant_from(agbuf_l, 0, keep_l, half)
        else:
            dequant_from(agbuf_r, s % 2, lax.rem(my - s + 1 + N_DEV, N_DEV), 0)
            dequant_from(agbuf_l, s % 2, lax.rem(my + s - 1 + N_DEV, N_DEV), half)
        r1.wait()
        r2.wait()
    dequant_from(agbuf_r, (N_DEV - 1) % 2, lax.rem(my - 2 + N_DEV, N_DEV), 0)
    dequant_from(agbuf_l, (N_DEV - 1) % 2, lax.rem(my + 2, N_DEV), half)


def kernel(x, w_mat):
    x = x.astype(jnp.bfloat16)
    w = w_mat.astype(jnp.bfloat16)
    m, _ = x.shape
    n = w.shape[1]
    chunk = m // N_DEV
    sub = chunk // SUB
    half = n // 2
    return pl.pallas_call(
        _body,
        out_shape=jax.ShapeDtypeStruct((m, n), jnp.float32),
        in_specs=[
            pl.BlockSpec(memory_space=pltpu.VMEM),
            pl.BlockSpec(memory_space=pltpu.VMEM),
        ],
        out_specs=pl.BlockSpec(memory_space=pltpu.VMEM),
        scratch_shapes=[
            pltpu.VMEM((SUB, sub, half), jnp.bfloat16),
            pltpu.VMEM((SUB, sub, half), jnp.bfloat16),
            pltpu.VMEM((SUB, sub, half), jnp.bfloat16),
            pltpu.VMEM((SUB, sub, half), jnp.bfloat16),
            pltpu.VMEM((2, chunk, half), jnp.float8_e4m3fn),
            pltpu.VMEM((2, chunk, half), jnp.float8_e4m3fn),
            pltpu.VMEM((N_DEV, 8, 128), jnp.float32),
            pltpu.SemaphoreType.DMA((N_DEV - 1, SUB)),
            pltpu.SemaphoreType.DMA((N_DEV - 1, SUB)),
            pltpu.SemaphoreType.DMA((N_DEV - 1, SUB)),
            pltpu.SemaphoreType.DMA((N_DEV - 1, SUB)),
            pltpu.SemaphoreType.DMA((N_DEV - 1,)),
            pltpu.SemaphoreType.DMA((N_DEV - 1,)),
            pltpu.SemaphoreType.DMA((N_DEV - 1,)),
            pltpu.SemaphoreType.DMA((N_DEV - 1,)),
            pltpu.SemaphoreType.DMA((N_DEV - 1,)),
            pltpu.SemaphoreType.DMA((N_DEV - 1,)),
            pltpu.SemaphoreType.REGULAR,
            pltpu.SemaphoreType.REGULAR,
        ],
        compiler_params=pltpu.CompilerParams(
            collective_id=0, vmem_limit_bytes=100 * 1024 * 1024
        ),
    )(x, w)


# device time: 180434 ns/iter; 3.5937x vs baseline; 1.0007x over previous
import jax
import jax.numpy as jnp
from jax import lax
from jax.experimental import pallas as pl
from jax.experimental.pallas import tpu as pltpu

N_DEV = 4
SUB = 4


def _body(
    x_ref,
    w_ref,
    out_ref,
    sendbuf_r,
    sendbuf_l,
    recvbuf_r,
    recvbuf_l,
    agbuf_r,
    agbuf_l,
    amax_buf,
    rsr_send,
    rsr_recv,
    rsl_send,
    rsl_recv,
    agr_send,
    agr_recv,
    agl_send,
    agl_recv,
    am_send,
    am_recv,
    credit_r,
    credit_l,
):
    m, n = out_ref.shape
    chunk = m // N_DEV
    sub = chunk // SUB
    half = n // 2
    my = lax.axis_index("i")
    left = lax.rem(my + N_DEV - 1, N_DEV)
    right = lax.rem(my + 1, N_DEV)

    barrier = pltpu.get_barrier_semaphore()
    for nbr in (left, right):
        pl.semaphore_signal(
            barrier, inc=1, device_id=(nbr,), device_id_type=pl.DeviceIdType.MESH
        )
    pl.semaphore_wait(barrier, 2)

    def gemm(c, c0):
        out_ref[pl.ds(c * chunk, chunk), pl.ds(c0, half)] = jnp.dot(
            x_ref[pl.ds(c * chunk, chunk), :],
            w_ref[:, pl.ds(c0, half)],
            preferred_element_type=jnp.float32,
        )

    gemm(my, 0)
    gemm(my, half)
    for s in range(N_DEV - 1):
        cr_s = lax.rem(my - s + N_DEV, N_DEV)
        cl_s = lax.rem(my + s, N_DEV)
        cr_n = lax.rem(my - s - 1 + N_DEV, N_DEV)
        cl_n = lax.rem(my + s + 1, N_DEV)
        if s >= 1:
            pl.semaphore_wait(credit_r, 1)
            pl.semaphore_wait(credit_l, 1)
        rr, rl = [], []
        for j in range(SUB):
            sendbuf_r[j] = out_ref[
                pl.ds(cr_s * chunk + j * sub, sub), pl.ds(0, half)
            ].astype(jnp.bfloat16)
            r = pltpu.make_async_remote_copy(
                src_ref=sendbuf_r.at[j],
                dst_ref=recvbuf_r.at[j],
                send_sem=rsr_send.at[s, j],
                recv_sem=rsr_recv.at[s, j],
                device_id=(right,),
                device_id_type=pl.DeviceIdType.MESH,
            )
            r.start()
            rr.append(r)
        for j in range(SUB):
            sendbuf_l[j] = out_ref[
                pl.ds(cl_s * chunk + j * sub, sub), pl.ds(half, half)
            ].astype(jnp.bfloat16)
            r = pltpu.make_async_remote_copy(
                src_ref=sendbuf_l.at[j],
                dst_ref=recvbuf_l.at[j],
                send_sem=rsl_send.at[s, j],
                recv_sem=rsl_recv.at[s, j],
                device_id=(left,),
                device_id_type=pl.DeviceIdType.MESH,
            )
            r.start()
            rl.append(r)
        gemm(cr_n, 0)
        gemm(cl_n, half)
        for j in range(SUB):
            rr[j].wait_recv()
            rows = pl.ds(cr_n * chunk + j * sub, sub)
            out_ref[rows, pl.ds(0, half)] += recvbuf_r[j].astype(jnp.float32)
        for j in range(SUB):
            rl[j].wait_recv()
            rows = pl.ds(cl_n * chunk + j * sub, sub)
            out_ref[rows, pl.ds(half, half)] += recvbuf_l[j].astype(jnp.float32)
        if s <= N_DEV - 3:
            pl.semaphore_signal(
                credit_r, inc=1, device_id=(left,), device_id_type=pl.DeviceIdType.MESH
            )
            pl.semaphore_signal(
                credit_l, inc=1, device_id=(right,), device_id_type=pl.DeviceIdType.MESH
            )
        for j in range(SUB):
            rr[j].wait_send()
            rl[j].wait_send()

    keep_r = lax.rem(my + 1, N_DEV)
    keep_l = lax.rem(my + N_DEV - 1, N_DEV)
    local_amax = jnp.maximum(
        jnp.max(jnp.abs(out_ref[pl.ds(keep_r * chunk, chunk), pl.ds(0, half)])),
        jnp.max(jnp.abs(out_ref[pl.ds(keep_l * chunk, chunk), pl.ds(half, half)])),
    )
    amax_buf[pl.ds(my, 1)] = jnp.full((1, 8, 128), local_amax, jnp.float32)
    am_rdmas = []
    for k in range(1, N_DEV):
        peer = lax.rem(my + k, N_DEV)
        r = pltpu.make_async_remote_copy(
            src_ref=amax_buf.at[pl.ds(my, 1)],
            dst_ref=amax_buf.at[pl.ds(my, 1)],
            send_sem=am_send.at[k - 1],
            recv_sem=am_recv.at[k - 1],
            device_id=(peer,),
            device_id_type=pl.DeviceIdType.MESH,
        )
        r.start()
        am_rdmas.append(r)
    for r in am_rdmas:
        r.wait_recv()
    g_amax = jnp.max(amax_buf[...])
    for r in am_rdmas:
        r.wait_send()
    scale = g_amax / 448.0
    inv_scale = 448.0 / g_amax

    def quant_to(buf, c, c0):
        for h in range(2):
            y = out_ref[pl.ds(c * chunk + h * (chunk // 2), chunk // 2), pl.ds(c0, half)]
            buf[0, pl.ds(h * (chunk // 2), chunk // 2), :] = jnp.clip(
                y * inv_scale, -448.0, 448.0
            ).astype(jnp.float8_e4m3fn)

    def dequant_from(buf, slot, c, c0):
        for h in range(2):
            rows = pl.ds(h * (chunk // 2), chunk // 2)
            out_ref[
                pl.ds(c * chunk + h * (chunk // 2), chunk // 2), pl.ds(c0, half)
            ] = buf[slot, rows, :].astype(jnp.float32) * scale

    quant_to(agbuf_r, keep_r, 0)
    quant_to(agbuf_l, keep_l, half)
    for s in range(N_DEV - 1):
        r1 = pltpu.make_async_remote_copy(
            src_ref=agbuf_r.at[s % 2],
            dst_ref=agbuf_r.at[(s + 1) % 2],
            send_sem=agr_send.at[s],
            recv_sem=agr_recv.at[s],
            device_id=(right,),
            device_id_type=pl.DeviceIdType.MESH,
        )
        r1.start()
        r2 = pltpu.make_async_remote_copy(
            src_ref=agbuf_l.at[s % 2],
            dst_ref=agbuf_l.at[(s + 1) % 2],
            send_sem=agl_send.at[s],
            recv_sem=agl_recv.at[s],
            device_id=(left,),
            device_id_type=pl.DeviceIdType.MESH,
        )
        r2.start()
        if s == 0:
            dequant_from(agbuf_r, 0, keep_r, 0)
            dequant_from(agbuf_l, 0, keep_l, half)
        else:
            dequant_from(agbuf_r, s % 2, lax.rem(my - s + 1 + N_DEV, N_DEV), 0)
            dequant_from(agbuf_l, s % 2, lax.rem(my + s - 1 + N_DEV, N_DEV), half)
        r1.wait()
        r2.wait()
    dequant_from(agbuf_r, (N_DEV - 1) % 2, lax.rem(my - 2 + N_DEV, N_DEV), 0)
    dequant_from(agbuf_l, (N_DEV - 1) % 2, lax.rem(my + 2, N_DEV), half)


def kernel(x, w_mat):
    x = x.astype(jnp.bfloat16)
    w = w_mat.astype(jnp.bfloat16)
    m, _ = x.shape
    n = w.shape[1]
    chunk = m // N_DEV
    sub = chunk // SUB
    half = n // 2
    return pl.pallas_call(
        _body,
        out_shape=jax.ShapeDtypeStruct((m, n), jnp.float32),
        in_specs=[
            pl.BlockSpec(memory_space=pltpu.VMEM),
            pl.BlockSpec(memory_space=pltpu.VMEM),
        ],
        out_specs=pl.BlockSpec(memory_space=pltpu.VMEM),
        scratch_shapes=[
            pltpu.VMEM((SUB, sub, half), jnp.bfloat16),
            pltpu.VMEM((SUB, sub, half), jnp.bfloat16),
            pltpu.VMEM((SUB, sub, half), jnp.bfloat16),
            pltpu.VMEM((SUB, sub, half), jnp.bfloat16),
            pltpu.VMEM((2, chunk, half), jnp.float8_e4m3fn),
            pltpu.VMEM((2, chunk, half), jnp.float8_e4m3fn),
            pltpu.VMEM((N_DEV, 8, 128), jnp.float32),
            pltpu.SemaphoreType.DMA((N_DEV - 1, SUB)),
            pltpu.SemaphoreType.DMA((N_DEV - 1, SUB)),
            pltpu.SemaphoreType.DMA((N_DEV - 1, SUB)),
            pltpu.SemaphoreType.DMA((N_DEV - 1, SUB)),
            pltpu.SemaphoreType.DMA((N_DEV - 1,)),
            pltpu.SemaphoreType.DMA((N_DEV - 1,)),
            pltpu.SemaphoreType.DMA((N_DEV - 1,)),
            pltpu.SemaphoreType.DMA((N_DEV - 1,)),
            pltpu.SemaphoreType.DMA((N_DEV - 1,)),
            pltpu.SemaphoreType.DMA((N_DEV - 1,)),
            pltpu.SemaphoreType.REGULAR,
            pltpu.SemaphoreType.REGULAR,
        ],
        compiler_params=pltpu.CompilerParams(
            collective_id=0, vmem_limit_bytes=100 * 1024 * 1024
        ),
    )(x, w)


# device time: 174584 ns/iter; 3.7142x vs baseline; 1.0335x over previous
import jax
import jax.numpy as jnp
from jax import lax
from jax.experimental import pallas as pl
from jax.experimental.pallas import tpu as pltpu

N_DEV = 4
SUB = 4


def _body(
    x_ref,
    w_ref,
    out_ref,
    sendbuf_r,
    sendbuf_l,
    recvbuf_r,
    recvbuf_l,
    agbuf_r,
    agbuf_l,
    amax_buf,
    rsr_send,
    rsr_recv,
    rsl_send,
    rsl_recv,
    agr_send,
    agr_recv,
    agl_send,
    agl_recv,
    am_send,
    am_recv,
    credit_r,
    credit_l,
):
    m, n = out_ref.shape
    chunk = m // N_DEV
    sub = chunk // SUB
    half = n // 2
    my = lax.axis_index("i")
    left = lax.rem(my + N_DEV - 1, N_DEV)
    right = lax.rem(my + 1, N_DEV)

    barrier = pltpu.get_barrier_semaphore()
    for nbr in (left, right):
        pl.semaphore_signal(
            barrier, inc=1, device_id=(nbr,), device_id_type=pl.DeviceIdType.MESH
        )
    pl.semaphore_wait(barrier, 2)

    def gemm(c, c0):
        out_ref[pl.ds(c * chunk, chunk), pl.ds(c0, half)] = jnp.dot(
            x_ref[pl.ds(c * chunk, chunk), :],
            w_ref[:, pl.ds(c0, half)],
            preferred_element_type=jnp.float32,
        )

    def rs_rdma(s, j, buf_s, buf_r, sems_s, sems_r, dev):
        return pltpu.make_async_remote_copy(
            src_ref=buf_s.at[j],
            dst_ref=buf_r.at[j],
            send_sem=sems_s.at[s, j],
            recv_sem=sems_r.at[s, j],
            device_id=(dev,),
            device_id_type=pl.DeviceIdType.MESH,
        )

    def cr_send(s):
        return lax.rem(my - s + N_DEV, N_DEV)

    def cl_send(s):
        return lax.rem(my + s, N_DEV)

    def stage_r(s, j):
        sendbuf_r[j] = out_ref[
            pl.ds(cr_send(s) * chunk + j * sub, sub), pl.ds(0, half)
        ].astype(jnp.bfloat16)
        r = rs_rdma(s, j, sendbuf_r, recvbuf_r, rsr_send, rsr_recv, right)
        r.start()
        return r

    def stage_l(s, j):
        sendbuf_l[j] = out_ref[
            pl.ds(cl_send(s) * chunk + j * sub, sub), pl.ds(half, half)
        ].astype(jnp.bfloat16)
        r = rs_rdma(s, j, sendbuf_l, recvbuf_l, rsl_send, rsl_recv, left)
        r.start()
        return r

    gemm(my, 0)
    gemm(my, half)
    rr = [stage_r(0, j) for j in range(SUB)]
    rl = [stage_l(0, j) for j in range(SUB)]
    gemm(lax.rem(my - 1 + N_DEV, N_DEV), 0)
    gemm(lax.rem(my + 1, N_DEV), half)
    for s in range(N_DEV - 1):
        cr_n = lax.rem(my - s - 1 + N_DEV, N_DEV)
        cl_n = lax.rem(my + s + 1, N_DEV)
        last = s == N_DEV - 2
        for j in range(SUB):
            rr[j].wait_recv()
            rows = pl.ds(cr_n * chunk + j * sub, sub)
            out_ref[rows, pl.ds(0, half)] += recvbuf_r[j].astype(jnp.float32)
            if not last:
                pl.semaphore_signal(
                    credit_r, inc=1, device_id=(left,),
                    device_id_type=pl.DeviceIdType.MESH,
                )
                pl.semaphore_wait(credit_r, 1)
                rr[j].wait_send()
                rr[j] = stage_r(s + 1, j)
            rl[j].wait_recv()
            rows = pl.ds(cl_n * chunk + j * sub, sub)
            out_ref[rows, pl.ds(half, half)] += recvbuf_l[j].astype(jnp.float32)
            if not last:
                pl.semaphore_signal(
                    credit_l, inc=1, device_id=(right,),
                    device_id_type=pl.DeviceIdType.MESH,
                )
                pl.semaphore_wait(credit_l, 1)
                rl[j].wait_send()
                rl[j] = stage_l(s + 1, j)
        if s == 0:
            gemm(lax.rem(my + 2, N_DEV), 0)
            gemm(lax.rem(my + 2, N_DEV), half)
        elif s == 1:
            gemm(lax.rem(my + 1, N_DEV), 0)
            gemm(lax.rem(my - 1 + N_DEV, N_DEV), half)
        if last:
            for j in range(SUB):
                rr[j].wait_send()
                rl[j].wait_send()

    keep_r = lax.rem(my + 1, N_DEV)
    keep_l = lax.rem(my + N_DEV - 1, N_DEV)
    local_amax = jnp.maximum(
        jnp.max(jnp.abs(out_ref[pl.ds(keep_r * chunk, chunk), pl.ds(0, half)])),
        jnp.max(jnp.abs(out_ref[pl.ds(keep_l * chunk, chunk), pl.ds(half, half)])),
    )
    amax_buf[pl.ds(my, 1)] = jnp.full((1, 8, 128), local_amax, jnp.float32)
    am_rdmas = []
    for k in range(1, N_DEV):
        peer = lax.rem(my + k, N_DEV)
        r = pltpu.make_async_remote_copy(
            src_ref=amax_buf.at[pl.ds(my, 1)],
            dst_ref=amax_buf.at[pl.ds(my, 1)],
            send_sem=am_send.at[k - 1],
            recv_sem=am_recv.at[k - 1],
            device_id=(peer,),
            device_id_type=pl.DeviceIdType.MESH,
        )
        r.start()
        am_rdmas.append(r)
    for r in am_rdmas:
        r.wait_recv()
    g_amax = jnp.max(amax_buf[...])
    for r in am_rdmas:
        r.wait_send()
    scale = g_amax / 448.0
    inv_scale = 448.0 / g_amax

    def quant_to(buf, c, c0):
        for h in range(2):
            y = out_ref[pl.ds(c * chunk + h * (chunk // 2), chunk // 2), pl.ds(c0, half)]
            buf[0, pl.ds(h * (chunk // 2), chunk // 2), :] = jnp.clip(
                y * inv_scale, -448.0, 448.0
            ).astype(jnp.float8_e4m3fn)

    def dequant_from(buf, slot, c, c0):
        for h in range(2):
            rows = pl.ds(h * (chunk // 2), chunk // 2)
            out_ref[
                pl.ds(c * chunk + h * (chunk // 2), chunk // 2), pl.ds(c0, half)
            ] = buf[slot, rows, :].astype(jnp.float32) * scale

    quant_to(agbuf_r, keep_r, 0)
    quant_to(agbuf_l, keep_l, half)
    for s in range(N_DEV - 1):
        r1 = pltpu.make_async_remote_copy(
            src_ref=agbuf_r.at[s % 2],
            dst_ref=agbuf_r.at[(s + 1) % 2],
            send_sem=agr_send.at[s],
            recv_sem=agr_recv.at[s],
            device_id=(right,),
            device_id_type=pl.DeviceIdType.MESH,
        )
        r1.start()
        r2 = pltpu.make_async_remote_copy(
            src_ref=agbuf_l.at[s % 2],
            dst_ref=agbuf_l.at[(s + 1) % 2],
            send_sem=agl_send.at[s],
            recv_sem=agl_recv.at[s],
            device_id=(left,),
            device_id_type=pl.DeviceIdType.MESH,
        )
        r2.start()
        if s == 0:
            dequant_from(agbuf_r, 0, keep_r, 0)
            dequant_from(agbuf_l, 0, keep_l, half)
        else:
            dequant_from(agbuf_r, s % 2, lax.rem(my - s + 1 + N_DEV, N_DEV), 0)
            dequant_from(agbuf_l, s % 2, lax.rem(my + s - 1 + N_DEV, N_DEV), half)
        r1.wait()
        r2.wait()
    dequant_from(agbuf_r, (N_DEV - 1) % 2, lax.rem(my - 2 + N_DEV, N_DEV), 0)
    dequant_from(agbuf_l, (N_DEV - 1) % 2, lax.rem(my + 2, N_DEV), half)


def kernel(x, w_mat):
    x = x.astype(jnp.bfloat16)
    w = w_mat.astype(jnp.bfloat16)
    m, _ = x.shape
    n = w.shape[1]
    chunk = m // N_DEV
    sub = chunk // SUB
    half = n // 2
    return pl.pallas_call(
        _body,
        out_shape=jax.ShapeDtypeStruct((m, n), jnp.float32),
        in_specs=[
            pl.BlockSpec(memory_space=pltpu.VMEM),
            pl.BlockSpec(memory_space=pltpu.VMEM),
        ],
        out_specs=pl.BlockSpec(memory_space=pltpu.VMEM),
        scratch_shapes=[
            pltpu.VMEM((SUB, sub, half), jnp.bfloat16),
            pltpu.VMEM((SUB, sub, half), jnp.bfloat16),
            pltpu.VMEM((SUB, sub, half), jnp.bfloat16),
            pltpu.VMEM((SUB, sub, half), jnp.bfloat16),
            pltpu.VMEM((2, chunk, half), jnp.float8_e4m3fn),
            pltpu.VMEM((2, chunk, half), jnp.float8_e4m3fn),
            pltpu.VMEM((N_DEV, 8, 128), jnp.float32),
            pltpu.SemaphoreType.DMA((N_DEV - 1, SUB)),
            pltpu.SemaphoreType.DMA((N_DEV - 1, SUB)),
            pltpu.SemaphoreType.DMA((N_DEV - 1, SUB)),
            pltpu.SemaphoreType.DMA((N_DEV - 1, SUB)),
            pltpu.SemaphoreType.DMA((N_DEV - 1,)),
            pltpu.SemaphoreType.DMA((N_DEV - 1,)),
            pltpu.SemaphoreType.DMA((N_DEV - 1,)),
            pltpu.SemaphoreType.DMA((N_DEV - 1,)),
            pltpu.SemaphoreType.DMA((N_DEV - 1,)),
            pltpu.SemaphoreType.DMA((N_DEV - 1,)),
            pltpu.SemaphoreType.REGULAR,
            pltpu.SemaphoreType.REGULAR,
        ],
        compiler_params=pltpu.CompilerParams(
            collective_id=0, vmem_limit_bytes=100 * 1024 * 1024
        ),
    )(x, w)


# device time: 166718 ns/iter; 3.8894x vs baseline; 1.0472x over previous
import jax
import jax.numpy as jnp
from jax import lax
from jax.experimental import pallas as pl
from jax.experimental.pallas import tpu as pltpu

N_DEV = 4
SUB = 4


def _body(
    x_ref,
    w_ref,
    out_ref,
    acc_ref,
    sendbuf_r,
    sendbuf_l,
    recvbuf_r,
    recvbuf_l,
    agbuf_r,
    agbuf_l,
    amax_buf,
    rsr_send,
    rsr_recv,
    rsl_send,
    rsl_recv,
    agr_send,
    agr_recv,
    agl_send,
    agl_recv,
    am_send,
    am_recv,
    credit_r,
    credit_l,
    store_sems,
):
    m, n = acc_ref.shape
    chunk = m // N_DEV
    sub = chunk // SUB
    half = n // 2
    my = lax.axis_index("i")
    left = lax.rem(my + N_DEV - 1, N_DEV)
    right = lax.rem(my + 1, N_DEV)

    barrier = pltpu.get_barrier_semaphore()
    for nbr in (left, right):
        pl.semaphore_signal(
            barrier, inc=1, device_id=(nbr,), device_id_type=pl.DeviceIdType.MESH
        )
    pl.semaphore_wait(barrier, 2)

    def gemm(c, c0):
        acc_ref[pl.ds(c * chunk, chunk), pl.ds(c0, half)] = jnp.dot(
            x_ref[pl.ds(c * chunk, chunk), :],
            w_ref[:, pl.ds(c0, half)],
            preferred_element_type=jnp.float32,
        )

    def rs_rdma(s, j, buf_s, buf_r, sems_s, sems_r, dev):
        return pltpu.make_async_remote_copy(
            src_ref=buf_s.at[j],
            dst_ref=buf_r.at[j],
            send_sem=sems_s.at[s, j],
            recv_sem=sems_r.at[s, j],
            device_id=(dev,),
            device_id_type=pl.DeviceIdType.MESH,
        )

    def cr_send(s):
        return lax.rem(my - s + N_DEV, N_DEV)

    def cl_send(s):
        return lax.rem(my + s, N_DEV)

    def stage_r(s, j):
        sendbuf_r[j] = acc_ref[
            pl.ds(cr_send(s) * chunk + j * sub, sub), pl.ds(0, half)
        ].astype(jnp.bfloat16)
        r = rs_rdma(s, j, sendbuf_r, recvbuf_r, rsr_send, rsr_recv, right)
        r.start()
        return r

    def stage_l(s, j):
        sendbuf_l[j] = acc_ref[
            pl.ds(cl_send(s) * chunk + j * sub, sub), pl.ds(half, half)
        ].astype(jnp.bfloat16)
        r = rs_rdma(s, j, sendbuf_l, recvbuf_l, rsl_send, rsl_recv, left)
        r.start()
        return r

    gemm(my, 0)
    gemm(my, half)
    rr = [stage_r(0, j) for j in range(SUB)]
    rl = [stage_l(0, j) for j in range(SUB)]
    gemm(lax.rem(my - 1 + N_DEV, N_DEV), 0)
    gemm(lax.rem(my + 1, N_DEV), half)
    for s in range(N_DEV - 1):
        cr_n = lax.rem(my - s - 1 + N_DEV, N_DEV)
        cl_n = lax.rem(my + s + 1, N_DEV)
        last = s == N_DEV - 2
        for j in range(SUB):
            rr[j].wait_recv()
            rows = pl.ds(cr_n * chunk + j * sub, sub)
            acc_ref[rows, pl.ds(0, half)] += recvbuf_r[j].astype(jnp.float32)
            if not last:
                pl.semaphore_signal(
                    credit_r, inc=1, device_id=(left,),
                    device_id_type=pl.DeviceIdType.MESH,
                )
                pl.semaphore_wait(credit_r, 1)
                rr[j].wait_send()
                rr[j] = stage_r(s + 1, j)
            rl[j].wait_recv()
            rows = pl.ds(cl_n * chunk + j * sub, sub)
            acc_ref[rows, pl.ds(half, half)] += recvbuf_l[j].astype(jnp.float32)
            if not last:
                pl.semaphore_signal(
                    credit_l, inc=1, device_id=(right,),
                    device_id_type=pl.DeviceIdType.MESH,
                )
                pl.semaphore_wait(credit_l, 1)
                rl[j].wait_send()
                rl[j] = stage_l(s + 1, j)
        if s == 0:
            gemm(lax.rem(my + 2, N_DEV), 0)
            gemm(lax.rem(my + 2, N_DEV), half)
        elif s == 1:
            gemm(lax.rem(my + 1, N_DEV), 0)
            gemm(lax.rem(my - 1 + N_DEV, N_DEV), half)
        if last:
            for j in range(SUB):
                rr[j].wait_send()
                rl[j].wait_send()

    keep_r = lax.rem(my + 1, N_DEV)
    keep_l = lax.rem(my + N_DEV - 1, N_DEV)
    local_amax = jnp.maximum(
        jnp.max(jnp.abs(acc_ref[pl.ds(keep_r * chunk, chunk), pl.ds(0, half)])),
        jnp.max(jnp.abs(acc_ref[pl.ds(keep_l * chunk, chunk), pl.ds(half, half)])),
    )
    amax_buf[pl.ds(my, 1)] = jnp.full((1, 8, 128), local_amax, jnp.float32)
    am_rdmas = []
    for k in range(1, N_DEV):
        peer = lax.rem(my + k, N_DEV)
        r = pltpu.make_async_remote_copy(
            src_ref=amax_buf.at[pl.ds(my, 1)],
            dst_ref=amax_buf.at[pl.ds(my, 1)],
            send_sem=am_send.at[k - 1],
            recv_sem=am_recv.at[k - 1],
            device_id=(peer,),
            device_id_type=pl.DeviceIdType.MESH,
        )
        r.start()
        am_rdmas.append(r)
    for r in am_rdmas:
        r.wait_recv()
    g_amax = jnp.max(amax_buf[...])
    for r in am_rdmas:
        r.wait_send()
    scale = g_amax / 448.0
    inv_scale = 448.0 / g_amax

    def quant_to(buf, c, c0):
        for h in range(2):
            y = acc_ref[pl.ds(c * chunk + h * (chunk // 2), chunk // 2), pl.ds(c0, half)]
            buf[0, pl.ds(h * (chunk // 2), chunk // 2), :] = jnp.clip(
                y * inv_scale, -448.0, 448.0
            ).astype(jnp.float8_e4m3fn)

    def dequant_from(buf, slot, c, c0, k):
        for h in range(2):
            rows = pl.ds(h * (chunk // 2), chunk // 2)
            acc_ref[
                pl.ds(c * chunk + h * (chunk // 2), chunk // 2), pl.ds(c0, half)
            ] = buf[slot, rows, :].astype(jnp.float32) * scale
        cp = pltpu.make_async_copy(
            acc_ref.at[pl.ds(c * chunk, chunk), pl.ds(c0, half)],
            out_ref.at[pl.ds(c * chunk, chunk), pl.ds(c0, half)],
            store_sems.at[k],
        )
        cp.start()
        return cp

    cps = []
    quant_to(agbuf_r, keep_r, 0)
    quant_to(agbuf_l, keep_l, half)
    for s in range(N_DEV - 1):
        r1 = pltpu.make_async_remote_copy(
            src_ref=agbuf_r.at[s % 2],
            dst_ref=agbuf_r.at[(s + 1) % 2],
            send_sem=agr_send.at[s],
            recv_sem=agr_recv.at[s],
            device_id=(right,),
            device_id_type=pl.DeviceIdType.MESH,
        )
        r1.start()
        r2 = pltpu.make_async_remote_copy(
            src_ref=agbuf_l.at[s % 2],
            dst_ref=agbuf_l.at[(s + 1) % 2],
            send_sem=agl_send.at[s],
            recv_sem=agl_recv.at[s],
            device_id=(left,),
            device_id_type=pl.DeviceIdType.MESH,
        )
        r2.start()
        if s == 0:
            cps.append(dequant_from(agbuf_r, 0, keep_r, 0, 0))
            cps.append(dequant_from(agbuf_l, 0, keep_l, half, 1))
        else:
            cps.append(
                dequant_from(agbuf_r, s % 2, lax.rem(my - s + 1 + N_DEV, N_DEV), 0, 2 * s)
            )
            cps.append(
                dequant_from(
                    agbuf_l, s % 2, lax.rem(my + s - 1 + N_DEV, N_DEV), half, 2 * s + 1
                )
            )
        r1.wait()
        r2.wait()
    cps.append(dequant_from(agbuf_r, (N_DEV - 1) % 2, lax.rem(my - 2 + N_DEV, N_DEV), 0, 6))
    cps.append(dequant_from(agbuf_l, (N_DEV - 1) % 2, lax.rem(my + 2, N_DEV), half, 7))
    for cp in cps:
        cp.wait()


def kernel(x, w_mat):
    x = x.astype(jnp.bfloat16)
    w = w_mat.astype(jnp.bfloat16)
    m, _ = x.shape
    n = w.shape[1]
    chunk = m // N_DEV
    sub = chunk // SUB
    half = n // 2
    return pl.pallas_call(
        _body,
        out_shape=jax.ShapeDtypeStruct((m, n), jnp.float32),
        in_specs=[
            pl.BlockSpec(memory_space=pltpu.VMEM),
            pl.BlockSpec(memory_space=pltpu.VMEM),
        ],
        out_specs=pl.BlockSpec(memory_space=pltpu.MemorySpace.HBM),
        scratch_shapes=[
            pltpu.VMEM((m, n), jnp.float32),
            pltpu.VMEM((SUB, sub, half), jnp.bfloat16),
            pltpu.VMEM((SUB, sub, half), jnp.bfloat16),
            pltpu.VMEM((SUB, sub, half), jnp.bfloat16),
            pltpu.VMEM((SUB, sub, half), jnp.bfloat16),
            pltpu.VMEM((2, chunk, half), jnp.float8_e4m3fn),
            pltpu.VMEM((2, chunk, half), jnp.float8_e4m3fn),
            pltpu.VMEM((N_DEV, 8, 128), jnp.float32),
            pltpu.SemaphoreType.DMA((N_DEV - 1, SUB)),
            pltpu.SemaphoreType.DMA((N_DEV - 1, SUB)),
            pltpu.SemaphoreType.DMA((N_DEV - 1, SUB)),
            pltpu.SemaphoreType.DMA((N_DEV - 1, SUB)),
            pltpu.SemaphoreType.DMA((N_DEV - 1,)),
            pltpu.SemaphoreType.DMA((N_DEV - 1,)),
            pltpu.SemaphoreType.DMA((N_DEV - 1,)),
            pltpu.SemaphoreType.DMA((N_DEV - 1,)),
            pltpu.SemaphoreType.DMA((N_DEV - 1,)),
            pltpu.SemaphoreType.DMA((N_DEV - 1,)),
            pltpu.SemaphoreType.REGULAR,
            pltpu.SemaphoreType.REGULAR,
            pltpu.SemaphoreType.DMA((2 * N_DEV,)),
        ],
        compiler_params=pltpu.CompilerParams(
            collective_id=0, vmem_limit_bytes=100 * 1024 * 1024
        ),
    )(x, w)


# device time: 155977 ns/iter; 4.1572x vs baseline; 1.0689x over previous
import jax
import jax.numpy as jnp
from jax import lax
from jax.experimental import pallas as pl
from jax.experimental.pallas import tpu as pltpu

N_DEV = 4
SUB = 4


def _body(
    x_ref,
    w_ref,
    out_ref,
    acc_ref,
    sendbuf_r,
    sendbuf_l,
    recvbuf_r,
    recvbuf_l,
    agbuf_r,
    agbuf_l,
    amax_buf,
    rsr_send,
    rsr_recv,
    rsl_send,
    rsl_recv,
    agr_send,
    agr_recv,
    agl_send,
    agl_recv,
    am_send,
    am_recv,
    credit_r,
    credit_l,
    str_sems,
    stl_sems,
):
    m, n = acc_ref.shape
    chunk = m // N_DEV
    sub = chunk // SUB
    half = n // 2
    my = lax.axis_index("i")
    left = lax.rem(my + N_DEV - 1, N_DEV)
    right = lax.rem(my + 1, N_DEV)

    barrier = pltpu.get_barrier_semaphore()
    for nbr in (left, right):
        pl.semaphore_signal(
            barrier, inc=1, device_id=(nbr,), device_id_type=pl.DeviceIdType.MESH
        )
    pl.semaphore_wait(barrier, 2)

    def gemm(c, c0):
        acc_ref[pl.ds(c * chunk, chunk), pl.ds(c0, half)] = jnp.dot(
            x_ref[pl.ds(c * chunk, chunk), :],
            w_ref[:, pl.ds(c0, half)],
            preferred_element_type=jnp.float32,
        )

    def rs_rdma(s, j, buf_s, buf_r, sems_s, sems_r, dev):
        return pltpu.make_async_remote_copy(
            src_ref=buf_s.at[j],
            dst_ref=buf_r.at[j],
            send_sem=sems_s.at[s, j],
            recv_sem=sems_r.at[s, j],
            device_id=(dev,),
            device_id_type=pl.DeviceIdType.MESH,
        )

    def cr_send(s):
        return lax.rem(my - s + N_DEV, N_DEV)

    def cl_send(s):
        return lax.rem(my + s, N_DEV)

    def stage_r(s, j):
        sendbuf_r[j] = acc_ref[
            pl.ds(cr_send(s) * chunk + j * sub, sub), pl.ds(0, half)
        ].astype(jnp.bfloat16)
        r = rs_rdma(s, j, sendbuf_r, recvbuf_r, rsr_send, rsr_recv, right)
        r.start()
        return r

    def stage_l(s, j):
        sendbuf_l[j] = acc_ref[
            pl.ds(cl_send(s) * chunk + j * sub, sub), pl.ds(half, half)
        ].astype(jnp.bfloat16)
        r = rs_rdma(s, j, sendbuf_l, recvbuf_l, rsl_send, rsl_recv, left)
        r.start()
        return r

    gemm(my, 0)
    gemm(my, half)
    rr = [stage_r(0, j) for j in range(SUB)]
    rl = [stage_l(0, j) for j in range(SUB)]
    gemm(lax.rem(my - 1 + N_DEV, N_DEV), 0)
    gemm(lax.rem(my + 1, N_DEV), half)
    for s in range(N_DEV - 1):
        cr_n = lax.rem(my - s - 1 + N_DEV, N_DEV)
        cl_n = lax.rem(my + s + 1, N_DEV)
        last = s == N_DEV - 2
        for j in range(SUB):
            rr[j].wait_recv()
            rows = pl.ds(cr_n * chunk + j * sub, sub)
            acc_ref[rows, pl.ds(0, half)] += recvbuf_r[j].astype(jnp.float32)
            if not last:
                pl.semaphore_signal(
                    credit_r, inc=1, device_id=(left,),
                    device_id_type=pl.DeviceIdType.MESH,
                )
                pl.semaphore_wait(credit_r, 1)
                rr[j].wait_send()
                rr[j] = stage_r(s + 1, j)
            rl[j].wait_recv()
            rows = pl.ds(cl_n * chunk + j * sub, sub)
            acc_ref[rows, pl.ds(half, half)] += recvbuf_l[j].astype(jnp.float32)
            if not last:
                pl.semaphore_signal(
                    credit_l, inc=1, device_id=(right,),
                    device_id_type=pl.DeviceIdType.MESH,
                )
                pl.semaphore_wait(credit_l, 1)
                rl[j].wait_send()
                rl[j] = stage_l(s + 1, j)
        if s == 0:
            gemm(lax.rem(my + 2, N_DEV), 0)
            gemm(lax.rem(my + 2, N_DEV), half)
        elif s == 1:
            gemm(lax.rem(my + 1, N_DEV), 0)
            gemm(lax.rem(my - 1 + N_DEV, N_DEV), half)
        if last:
            for j in range(SUB):
                rr[j].wait_send()
                rl[j].wait_send()

    keep_r = lax.rem(my + 1, N_DEV)
    keep_l = lax.rem(my + N_DEV - 1, N_DEV)
    local_amax = jnp.maximum(
        jnp.max(jnp.abs(acc_ref[pl.ds(keep_r * chunk, chunk), pl.ds(0, half)])),
        jnp.max(jnp.abs(acc_ref[pl.ds(keep_l * chunk, chunk), pl.ds(half, half)])),
    )
    amax_buf[pl.ds(my, 1)] = jnp.full((1, 8, 128), local_amax, jnp.float32)
    am_rdmas = []
    for k in range(1, N_DEV):
        peer = lax.rem(my + k, N_DEV)
        r = pltpu.make_async_remote_copy(
            src_ref=amax_buf.at[pl.ds(my, 1)],
            dst_ref=amax_buf.at[pl.ds(my, 1)],
            send_sem=am_send.at[k - 1],
            recv_sem=am_recv.at[k - 1],
            device_id=(peer,),
            device_id_type=pl.DeviceIdType.MESH,
        )
        r.start()
        am_rdmas.append(r)
    for r in am_rdmas:
        r.wait_recv()
    g_amax = jnp.max(amax_buf[...])
    for r in am_rdmas:
        r.wait_send()
    scale = g_amax / 448.0
    inv_scale = 448.0 / g_amax

    def quant_to(buf, c, c0):
        for h in range(2):
            y = acc_ref[pl.ds(c * chunk + h * (chunk // 2), chunk // 2), pl.ds(c0, half)]
            buf[0, pl.ds(h * (chunk // 2), chunk // 2), :] = jnp.clip(
                y * inv_scale, -448.0, 448.0
            ).astype(jnp.float8_e4m3fn)

    def dequant_store(buf, slot, c, c0, sbuf, ssems, prev):
        cur = []
        for j in range(SUB):
            if prev is not None:
                prev[j].wait()
            sbuf[j] = (
                buf[slot, pl.ds(j * sub, sub), :].astype(jnp.float32) * scale
            ).astype(jnp.bfloat16)
            cp = pltpu.make_async_copy(
                sbuf.at[j],
                out_ref.at[pl.ds(c * chunk + j * sub, sub), pl.ds(c0, half)],
                ssems.at[j],
            )
            cp.start()
            cur.append(cp)
        return cur

    prev_r = prev_l = None
    quant_to(agbuf_r, keep_r, 0)
    quant_to(agbuf_l, keep_l, half)
    for s in range(N_DEV - 1):
        r1 = pltpu.make_async_remote_copy(
            src_ref=agbuf_r.at[s % 2],
            dst_ref=agbuf_r.at[(s + 1) % 2],
            send_sem=agr_send.at[s],
            recv_sem=agr_recv.at[s],
            device_id=(right,),
            device_id_type=pl.DeviceIdType.MESH,
        )
        r1.start()
        r2 = pltpu.make_async_remote_copy(
            src_ref=agbuf_l.at[s % 2],
            dst_ref=agbuf_l.at[(s + 1) % 2],
            send_sem=agl_send.at[s],
            recv_sem=agl_recv.at[s],
            device_id=(left,),
            device_id_type=pl.DeviceIdType.MESH,
        )
        r2.start()
        if s == 0:
            prev_r = dequant_store(agbuf_r, 0, keep_r, 0, sendbuf_r, str_sems, prev_r)
            prev_l = dequant_store(agbuf_l, 0, keep_l, half, sendbuf_l, stl_sems, prev_l)
        else:
            prev_r = dequant_store(
                agbuf_r, s % 2, lax.rem(my - s + 1 + N_DEV, N_DEV), 0,
                sendbuf_r, str_sems, prev_r,
            )
            prev_l = dequant_store(
                agbuf_l, s % 2, lax.rem(my + s - 1 + N_DEV, N_DEV), half,
                sendbuf_l, stl_sems, prev_l,
            )
        r1.wait()
        r2.wait()
    prev_r = dequant_store(
        agbuf_r, (N_DEV - 1) % 2, lax.rem(my - 2 + N_DEV, N_DEV), 0,
        sendbuf_r, str_sems, prev_r,
    )
    prev_l = dequant_store(
        agbuf_l, (N_DEV - 1) % 2, lax.rem(my + 2, N_DEV), half,
        sendbuf_l, stl_sems, prev_l,
    )
    for cp in prev_r + prev_l:
        cp.wait()


def kernel(x, w_mat):
    x = x.astype(jnp.bfloat16)
    w = w_mat.astype(jnp.bfloat16)
    m, _ = x.shape
    n = w.shape[1]
    chunk = m // N_DEV
    sub = chunk // SUB
    half = n // 2
    return pl.pallas_call(
        _body,
        out_shape=jax.ShapeDtypeStruct((m, n), jnp.bfloat16),
        in_specs=[
            pl.BlockSpec(memory_space=pltpu.VMEM),
            pl.BlockSpec(memory_space=pltpu.VMEM),
        ],
        out_specs=pl.BlockSpec(memory_space=pltpu.MemorySpace.HBM),
        scratch_shapes=[
            pltpu.VMEM((m, n), jnp.float32),
            pltpu.VMEM((SUB, sub, half), jnp.bfloat16),
            pltpu.VMEM((SUB, sub, half), jnp.bfloat16),
            pltpu.VMEM((SUB, sub, half), jnp.bfloat16),
            pltpu.VMEM((SUB, sub, half), jnp.bfloat16),
            pltpu.VMEM((2, chunk, half), jnp.float8_e4m3fn),
            pltpu.VMEM((2, chunk, half), jnp.float8_e4m3fn),
            pltpu.VMEM((N_DEV, 8, 128), jnp.float32),
            pltpu.SemaphoreType.DMA((N_DEV - 1, SUB)),
            pltpu.SemaphoreType.DMA((N_DEV - 1, SUB)),
            pltpu.SemaphoreType.DMA((N_DEV - 1, SUB)),
            pltpu.SemaphoreType.DMA((N_DEV - 1, SUB)),
            pltpu.SemaphoreType.DMA((N_DEV - 1,)),
            pltpu.SemaphoreType.DMA((N_DEV - 1,)),
            pltpu.SemaphoreType.DMA((N_DEV - 1,)),
            pltpu.SemaphoreType.DMA((N_DEV - 1,)),
            pltpu.SemaphoreType.DMA((N_DEV - 1,)),
            pltpu.SemaphoreType.DMA((N_DEV - 1,)),
            pltpu.SemaphoreType.REGULAR,
            pltpu.SemaphoreType.REGULAR,
            pltpu.SemaphoreType.DMA((SUB,)),
            pltpu.SemaphoreType.DMA((SUB,)),
        ],
        compiler_params=pltpu.CompilerParams(
            collective_id=0, vmem_limit_bytes=100 * 1024 * 1024
        ),
    )(x, w)


# device time: 146095 ns/iter; 4.4384x vs baseline; 1.0676x over previous
import jax
import jax.numpy as jnp
from jax import lax
from jax.experimental import pallas as pl
from jax.experimental.pallas import tpu as pltpu

N_DEV = 4
SUB = 4


def _body(
    x_ref,
    w_ref,
    out_ref,
    acc_ref,
    xs_ref,
    xb_ref,
    sendbuf_r,
    sendbuf_l,
    recvbuf_r,
    recvbuf_l,
    agbuf_r,
    agbuf_l,
    amax_buf,
    rsr_send,
    rsr_recv,
    rsl_send,
    rsl_recv,
    agr_send,
    agr_recv,
    agl_send,
    agl_recv,
    am_send,
    am_recv,
    credit_r,
    credit_l,
    str_sems,
    stl_sems,
    x_sem,
):
    m, n = acc_ref.shape
    chunk = m // N_DEV
    sub = chunk // SUB
    half = n // 2
    my = lax.axis_index("i")
    left = lax.rem(my + N_DEV - 1, N_DEV)
    right = lax.rem(my + 1, N_DEV)

    barrier = pltpu.get_barrier_semaphore()
    for nbr in (left, right):
        pl.semaphore_signal(
            barrier, inc=1, device_id=(nbr,), device_id_type=pl.DeviceIdType.MESH
        )
    pl.semaphore_wait(barrier, 2)

    def start_load(c):
        cp = pltpu.make_async_copy(
            x_ref.at[pl.ds(c * chunk, chunk), :], xs_ref.at[0], x_sem
        )
        cp.start()
        return cp

    def finish_load(cp, slot):
        cp.wait()
        xb_ref[slot] = xs_ref[0].astype(jnp.bfloat16)

    def gemm(slot, c, c0):
        acc_ref[pl.ds(c * chunk, chunk), pl.ds(c0, half)] = jnp.dot(
            xb_ref[slot],
            w_ref[:, pl.ds(c0, half)],
            preferred_element_type=jnp.float32,
        )

    def rs_rdma(s, j, buf_s, buf_r, sems_s, sems_r, dev):
        return pltpu.make_async_remote_copy(
            src_ref=buf_s.at[j],
            dst_ref=buf_r.at[j],
            send_sem=sems_s.at[s, j],
            recv_sem=sems_r.at[s, j],
            device_id=(dev,),
            device_id_type=pl.DeviceIdType.MESH,
        )

    def cr_send(s):
        return lax.rem(my - s + N_DEV, N_DEV)

    def cl_send(s):
        return lax.rem(my + s, N_DEV)

    def stage_r(s, j):
        sendbuf_r[j] = acc_ref[
            pl.ds(cr_send(s) * chunk + j * sub, sub), pl.ds(0, half)
        ].astype(jnp.bfloat16)
        r = rs_rdma(s, j, sendbuf_r, recvbuf_r, rsr_send, rsr_recv, right)
        r.start()
        return r

    def stage_l(s, j):
        sendbuf_l[j] = acc_ref[
            pl.ds(cl_send(s) * chunk + j * sub, sub), pl.ds(half, half)
        ].astype(jnp.bfloat16)
        r = rs_rdma(s, j, sendbuf_l, recvbuf_l, rsl_send, rsl_recv, left)
        r.start()
        return r

    c_m1 = lax.rem(my - 1 + N_DEV, N_DEV)
    c_p1 = lax.rem(my + 1, N_DEV)
    c_p2 = lax.rem(my + 2, N_DEV)
    cp = start_load(my)
    finish_load(cp, 0)
    gemm(0, my, 0)
    gemm(0, my, half)
    cp = start_load(c_m1)
    rr = [stage_r(0, j) for j in range(SUB)]
    rl = [stage_l(0, j) for j in range(SUB)]
    finish_load(cp, 1)
    gemm(1, c_m1, 0)
    cp = start_load(c_p1)
    finish_load(cp, 0)
    gemm(0, c_p1, half)
    cp = start_load(c_p2)
    for s in range(N_DEV - 1):
        cr_n = lax.rem(my - s - 1 + N_DEV, N_DEV)
        cl_n = lax.rem(my + s + 1, N_DEV)
        last = s == N_DEV - 2
        for j in range(SUB):
            rr[j].wait_recv()
            rows = pl.ds(cr_n * chunk + j * sub, sub)
            acc_ref[rows, pl.ds(0, half)] += recvbuf_r[j].astype(jnp.float32)
            if not last:
                pl.semaphore_signal(
                    credit_r, inc=1, device_id=(left,),
                    device_id_type=pl.DeviceIdType.MESH,
                )
                pl.semaphore_wait(credit_r, 1)
                rr[j].wait_send()
                rr[j] = stage_r(s + 1, j)
            rl[j].wait_recv()
            rows = pl.ds(cl_n * chunk + j * sub, sub)
            acc_ref[rows, pl.ds(half, half)] += recvbuf_l[j].astype(jnp.float32)
            if not last:
                pl.semaphore_signal(
                    credit_l, inc=1, device_id=(right,),
                    device_id_type=pl.DeviceIdType.MESH,
                )
                pl.semaphore_wait(credit_l, 1)
                rl[j].wait_send()
                rl[j] = stage_l(s + 1, j)
        if s == 0:
            finish_load(cp, 1)
            gemm(1, c_p2, 0)
            gemm(1, c_p2, half)
            cp = start_load(c_m1)
        elif s == 1:
            gemm(0, c_p1, 0)
            finish_load(cp, 1)
            gemm(1, c_m1, half)
        if last:
            for j in range(SUB):
                rr[j].wait_send()
                rl[j].wait_send()

    keep_r = lax.rem(my + 1, N_DEV)
    keep_l = lax.rem(my + N_DEV - 1, N_DEV)
    local_amax = jnp.maximum(
        jnp.max(jnp.abs(acc_ref[pl.ds(keep_r * chunk, chunk), pl.ds(0, half)])),
        jnp.max(jnp.abs(acc_ref[pl.ds(keep_l * chunk, chunk), pl.ds(half, half)])),
    )
    amax_buf[pl.ds(my, 1)] = jnp.full((1, 8, 128), local_amax, jnp.float32)
    am_rdmas = []
    for k in range(1, N_DEV):
        peer = lax.rem(my + k, N_DEV)
        r = pltpu.make_async_remote_copy(
            src_ref=amax_buf.at[pl.ds(my, 1)],
            dst_ref=amax_buf.at[pl.ds(my, 1)],
            send_sem=am_send.at[k - 1],
            recv_sem=am_recv.at[k - 1],
            device_id=(peer,),
            device_id_type=pl.DeviceIdType.MESH,
        )
        r.start()
        am_rdmas.append(r)
    for r in am_rdmas:
        r.wait_recv()
    g_amax = jnp.max(amax_buf[...])
    for r in am_rdmas:
        r.wait_send()
    scale = g_amax / 448.0
    inv_scale = 448.0 / g_amax

    def quant_to(buf, c, c0):
        for h in range(2):
            y = acc_ref[pl.ds(c * chunk + h * (chunk // 2), chunk // 2), pl.ds(c0, half)]
            buf[0, pl.ds(h * (chunk // 2), chunk // 2), :] = jnp.clip(
                y * inv_scale, -448.0, 448.0
            ).astype(jnp.float8_e4m3fn)

    def dequant_store(buf, slot, c, c0, sbuf, ssems, prev):
        cur = []
        for j in range(SUB):
            if prev is not None:
                prev[j].wait()
            sbuf[j] = (
                buf[slot, pl.ds(j * sub, sub), :].astype(jnp.float32) * scale
            ).astype(jnp.bfloat16)
            cp = pltpu.make_async_copy(
                sbuf.at[j],
                out_ref.at[pl.ds(c * chunk + j * sub, sub), pl.ds(c0, half)],
                ssems.at[j],
            )
            cp.start()
            cur.append(cp)
        return cur

    prev_r = prev_l = None
    quant_to(agbuf_r, keep_r, 0)
    quant_to(agbuf_l, keep_l, half)
    for s in range(N_DEV - 1):
        r1 = pltpu.make_async_remote_copy(
            src_ref=agbuf_r.at[s % 2],
            dst_ref=agbuf_r.at[(s + 1) % 2],
            send_sem=agr_send.at[s],
            recv_sem=agr_recv.at[s],
            device_id=(right,),
            device_id_type=pl.DeviceIdType.MESH,
        )
        r1.start()
        r2 = pltpu.make_async_remote_copy(
            src_ref=agbuf_l.at[s % 2],
            dst_ref=agbuf_l.at[(s + 1) % 2],
            send_sem=agl_send.at[s],
            recv_sem=agl_recv.at[s],
            device_id=(left,),
            device_id_type=pl.DeviceIdType.MESH,
        )
        r2.start()
        if s == 0:
            prev_r = dequant_store(agbuf_r, 0, keep_r, 0, sendbuf_r, str_sems, prev_r)
            prev_l = dequant_store(agbuf_l, 0, keep_l, half, sendbuf_l, stl_sems, prev_l)
        else:
            prev_r = dequant_store(
                agbuf_r, s % 2, lax.rem(my - s + 1 + N_DEV, N_DEV), 0,
                sendbuf_r, str_sems, prev_r,
            )
            prev_l = dequant_store(
                agbuf_l, s % 2, lax.rem(my + s - 1 + N_DEV, N_DEV), half,
                sendbuf_l, stl_sems, prev_l,
            )
        r1.wait()
        r2.wait()
    prev_r = dequant_store(
        agbuf_r, (N_DEV - 1) % 2, lax.rem(my - 2 + N_DEV, N_DEV), 0,
        sendbuf_r, str_sems, prev_r,
    )
    prev_l = dequant_store(
        agbuf_l, (N_DEV - 1) % 2, lax.rem(my + 2, N_DEV), half,
        sendbuf_l, stl_sems, prev_l,
    )
    for cp in prev_r + prev_l:
        cp.wait()


def kernel(x, w_mat):
    w = w_mat.astype(jnp.bfloat16)
    m, k_per = x.shape
    n = w.shape[1]
    chunk = m // N_DEV
    sub = chunk // SUB
    half = n // 2
    return pl.pallas_call(
        _body,
        out_shape=jax.ShapeDtypeStruct((m, n), jnp.bfloat16),
        in_specs=[
            pl.BlockSpec(memory_space=pltpu.MemorySpace.HBM),
            pl.BlockSpec(memory_space=pltpu.VMEM),
        ],
        out_specs=pl.BlockSpec(memory_space=pltpu.MemorySpace.HBM),
        scratch_shapes=[
            pltpu.VMEM((m, n), jnp.float32),
            pltpu.VMEM((1, chunk, k_per), jnp.float32),
            pltpu.VMEM((2, chunk, k_per), jnp.bfloat16),
            pltpu.VMEM((SUB, sub, half), jnp.bfloat16),
            pltpu.VMEM((SUB, sub, half), jnp.bfloat16),
            pltpu.VMEM((SUB, sub, half), jnp.bfloat16),
            pltpu.VMEM((SUB, sub, half), jnp.bfloat16),
            pltpu.VMEM((2, chunk, half), jnp.float8_e4m3fn),
            pltpu.VMEM((2, chunk, half), jnp.float8_e4m3fn),
            pltpu.VMEM((N_DEV, 8, 128), jnp.float32),
            pltpu.SemaphoreType.DMA((N_DEV - 1, SUB)),
            pltpu.SemaphoreType.DMA((N_DEV - 1, SUB)),
            pltpu.SemaphoreType.DMA((N_DEV - 1, SUB)),
            pltpu.SemaphoreType.DMA((N_DEV - 1, SUB)),
            pltpu.SemaphoreType.DMA((N_DEV - 1,)),
            pltpu.SemaphoreType.DMA((N_DEV - 1,)),
            pltpu.SemaphoreType.DMA((N_DEV - 1,)),
            pltpu.SemaphoreType.DMA((N_DEV - 1,)),
            pltpu.SemaphoreType.DMA((N_DEV - 1,)),
            pltpu.SemaphoreType.DMA((N_DEV - 1,)),
            pltpu.SemaphoreType.REGULAR,
            pltpu.SemaphoreType.REGULAR,
            pltpu.SemaphoreType.DMA((SUB,)),
            pltpu.SemaphoreType.DMA((SUB,)),
            pltpu.SemaphoreType.DMA,
        ],
        compiler_params=pltpu.CompilerParams(
            collective_id=0, vmem_limit_bytes=100 * 1024 * 1024
        ),
    )(x, w)


# device time: 142337 ns/iter; 4.5556x vs baseline; 1.0264x over previous
import jax
import jax.numpy as jnp
from jax import lax
from jax.experimental import pallas as pl
from jax.experimental.pallas import tpu as pltpu

N_DEV = 4
SUB = 4


def _body(
    x_ref,
    w_ref,
    out_ref,
    acc_ref,
    xs_ref,
    xb_ref,
    sendbuf_r,
    sendbuf_l,
    recvbuf_r,
    recvbuf_l,
    agbuf_r,
    agbuf_l,
    amax_buf,
    rsr_send,
    rsr_recv,
    rsl_send,
    rsl_recv,
    agr_send,
    agr_recv,
    agl_send,
    agl_recv,
    am_send,
    am_recv,
    credit_r,
    credit_l,
    str_sems,
    stl_sems,
    x_sem,
):
    m, n = acc_ref.shape
    chunk = m // N_DEV
    sub = chunk // SUB
    half = n // 2
    my = lax.axis_index("i")
    left = lax.rem(my + N_DEV - 1, N_DEV)
    right = lax.rem(my + 1, N_DEV)

    barrier = pltpu.get_barrier_semaphore()
    for nbr in (left, right):
        pl.semaphore_signal(
            barrier, inc=1, device_id=(nbr,), device_id_type=pl.DeviceIdType.MESH
        )
    pl.semaphore_wait(barrier, 2)

    def start_load(c):
        cp = pltpu.make_async_copy(
            x_ref.at[pl.ds(c * chunk, chunk), :], xs_ref.at[0], x_sem
        )
        cp.start()
        return cp

    def finish_load(cp, slot):
        cp.wait()
        xb_ref[slot] = xs_ref[0].astype(jnp.bfloat16)

    def gemm(slot, c, c0):
        acc_ref[pl.ds(c * chunk, chunk), pl.ds(c0, half)] = jnp.dot(
            xb_ref[slot],
            w_ref[:, pl.ds(c0, half)],
            preferred_element_type=jnp.float32,
        )

    def rs_rdma(s, j, buf_s, buf_r, sems_s, sems_r, dev):
        return pltpu.make_async_remote_copy(
            src_ref=buf_s.at[j],
            dst_ref=buf_r.at[j],
            send_sem=sems_s.at[s, j],
            recv_sem=sems_r.at[s, j],
            device_id=(dev,),
            device_id_type=pl.DeviceIdType.MESH,
        )

    def cr_send(s):
        return lax.rem(my - s + N_DEV, N_DEV)

    def cl_send(s):
        return lax.rem(my + s, N_DEV)

    def stage_r(s, j):
        sendbuf_r[j] = acc_ref[
            pl.ds(cr_send(s) * chunk + j * sub, sub), pl.ds(0, half)
        ].astype(jnp.bfloat16)
        r = rs_rdma(s, j, sendbuf_r, recvbuf_r, rsr_send, rsr_recv, right)
        r.start()
        return r

    def stage_l(s, j):
        sendbuf_l[j] = acc_ref[
            pl.ds(cl_send(s) * chunk + j * sub, sub), pl.ds(half, half)
        ].astype(jnp.bfloat16)
        r = rs_rdma(s, j, sendbuf_l, recvbuf_l, rsl_send, rsl_recv, left)
        r.start()
        return r

    c_m1 = lax.rem(my - 1 + N_DEV, N_DEV)
    c_p1 = lax.rem(my + 1, N_DEV)
    c_p2 = lax.rem(my + 2, N_DEV)
    cp = start_load(my)
    finish_load(cp, 0)
    gemm(0, my, 0)
    gemm(0, my, half)
    cp = start_load(c_m1)
    rr = [stage_r(0, j) for j in range(SUB)]
    rl = [stage_l(0, j) for j in range(SUB)]
    finish_load(cp, 1)
    gemm(1, c_m1, 0)
    cp = start_load(c_p1)
    finish_load(cp, 0)
    gemm(0, c_p1, half)
    cp = start_load(c_p2)
    for s in range(N_DEV - 1):
        cr_n = lax.rem(my - s - 1 + N_DEV, N_DEV)
        cl_n = lax.rem(my + s + 1, N_DEV)
        last = s == N_DEV - 2
        for j in range(SUB):
            rr[j].wait_recv()
            rows = pl.ds(cr_n * chunk + j * sub, sub)
            acc_ref[rows, pl.ds(0, half)] += recvbuf_r[j].astype(jnp.float32)
            if not last:
                pl.semaphore_signal(
                    credit_r, inc=1, device_id=(left,),
                    device_id_type=pl.DeviceIdType.MESH,
                )
                pl.semaphore_wait(credit_r, 1)
                rr[j].wait_send()
                rr[j] = stage_r(s + 1, j)
            rl[j].wait_recv()
            rows = pl.ds(cl_n * chunk + j * sub, sub)
            acc_ref[rows, pl.ds(half, half)] += recvbuf_l[j].astype(jnp.float32)
            if not last:
                pl.semaphore_signal(
                    credit_l, inc=1, device_id=(right,),
                    device_id_type=pl.DeviceIdType.MESH,
                )
                pl.semaphore_wait(credit_l, 1)
                rl[j].wait_send()
                rl[j] = stage_l(s + 1, j)
        if s == 0:
            finish_load(cp, 1)
            gemm(1, c_p2, 0)
            gemm(1, c_p2, half)
            cp = start_load(c_m1)
        elif s == 1:
            gemm(0, c_p1, 0)
            finish_load(cp, 1)
            gemm(1, c_m1, half)
        if last:
            for j in range(SUB):
                rr[j].wait_send()
                rl[j].wait_send()

    keep_r = lax.rem(my + 1, N_DEV)
    keep_l = lax.rem(my + N_DEV - 1, N_DEV)
    local_amax = jnp.maximum(
        jnp.max(jnp.abs(acc_ref[pl.ds(keep_r * chunk, chunk), pl.ds(0, half)])),
        jnp.max(jnp.abs(acc_ref[pl.ds(keep_l * chunk, chunk), pl.ds(half, half)])),
    )
    amax_buf[pl.ds(my, 1)] = jnp.full((1, 8, 128), local_amax, jnp.float32)
    am_rdmas = []
    for k in range(1, N_DEV):
        peer = lax.rem(my + k, N_DEV)
        r = pltpu.make_async_remote_copy(
            src_ref=amax_buf.at[pl.ds(my, 1)],
            dst_ref=amax_buf.at[pl.ds(my, 1)],
            send_sem=am_send.at[k - 1],
            recv_sem=am_recv.at[k - 1],
            device_id=(peer,),
            device_id_type=pl.DeviceIdType.MESH,
        )
        r.start()
        am_rdmas.append(r)
    for r in am_rdmas:
        r.wait_recv()
    g_amax = jnp.max(amax_buf[...])
    for r in am_rdmas:
        r.wait_send()
    scale = g_amax / 448.0
    inv_scale = 448.0 / g_amax

    def quant_piece(buf, c, c0, j):
        y = acc_ref[pl.ds(c * chunk + j * sub, sub), pl.ds(c0, half)]
        buf[0, pl.ds(j * sub, sub), :] = jnp.clip(
            y * inv_scale, -448.0, 448.0
        ).astype(jnp.float8_e4m3fn)

    def dequant_store(buf, slot, c, c0, sbuf, ssems, prev):
        cur = []
        for j in range(SUB):
            if prev is not None:
                prev[j].wait()
            sbuf[j] = (
                buf[slot, pl.ds(j * sub, sub), :].astype(jnp.float32) * scale
            ).astype(jnp.bfloat16)
            cp = pltpu.make_async_copy(
                sbuf.at[j],
                out_ref.at[pl.ds(c * chunk + j * sub, sub), pl.ds(c0, half)],
                ssems.at[j],
            )
            cp.start()
            cur.append(cp)
        return cur

    def ag_send(buf, sems_s, sems_r, dev, s, j):
        r = pltpu.make_async_remote_copy(
            src_ref=buf.at[s % 2, pl.ds(j * sub, sub), :],
            dst_ref=buf.at[(s + 1) % 2, pl.ds(j * sub, sub), :],
            send_sem=sems_s.at[s, j],
            recv_sem=sems_r.at[s, j],
            device_id=(dev,),
            device_id_type=pl.DeviceIdType.MESH,
        )
        r.start()
        return r

    ag_r, ag_l = {}, {}
    for j in range(SUB):
        quant_piece(agbuf_r, keep_r, 0, j)
        ag_r[(0, j)] = ag_send(agbuf_r, agr_send, agr_recv, right, 0, j)
        quant_piece(agbuf_l, keep_l, half, j)
        ag_l[(0, j)] = ag_send(agbuf_l, agl_send, agl_recv, left, 0, j)
    prev_r = dequant_store(agbuf_r, 0, keep_r, 0, sendbuf_r, str_sems, None)
    prev_l = dequant_store(agbuf_l, 0, keep_l, half, sendbuf_l, stl_sems, None)
    for s in range(N_DEV - 1):
        for j in range(SUB):
            ag_r[(s, j)].wait_recv()
            if s < N_DEV - 2:
                ag_r[(s + 1, j)] = ag_send(agbuf_r, agr_send, agr_recv, right, s + 1, j)
            ag_l[(s, j)].wait_recv()
            if s < N_DEV - 2:
                ag_l[(s + 1, j)] = ag_send(agbuf_l, agl_send, agl_recv, left, s + 1, j)
        prev_r = dequant_store(
            agbuf_r, (s + 1) % 2, lax.rem(my - s + N_DEV, N_DEV), 0,
            sendbuf_r, str_sems, prev_r,
        )
        prev_l = dequant_store(
            agbuf_l, (s + 1) % 2, lax.rem(my + s, N_DEV), half,
            sendbuf_l, stl_sems, prev_l,
        )
    for r in ag_r.values():
        r.wait_send()
    for r in ag_l.values():
        r.wait_send()
    for cp in prev_r + prev_l:
        cp.wait()


def kernel(x, w_mat):
    w = w_mat.astype(jnp.bfloat16)
    m, k_per = x.shape
    n = w.shape[1]
    chunk = m // N_DEV
    sub = chunk // SUB
    half = n // 2
    return pl.pallas_call(
        _body,
        out_shape=jax.ShapeDtypeStruct((m, n), jnp.bfloat16),
        in_specs=[
            pl.BlockSpec(memory_space=pltpu.MemorySpace.HBM),
            pl.BlockSpec(memory_space=pltpu.VMEM),
        ],
        out_specs=pl.BlockSpec(memory_space=pltpu.MemorySpace.HBM),
        scratch_shapes=[
            pltpu.VMEM((m, n), jnp.float32),
            pltpu.VMEM((1, chunk, k_per), jnp.float32),
            pltpu.VMEM((2, chunk, k_per), jnp.bfloat16),
            pltpu.VMEM((SUB, sub, half), jnp.bfloat16),
            pltpu.VMEM((SUB, sub, half), jnp.bfloat16),
            pltpu.VMEM((SUB, sub, half), jnp.bfloat16),
            pltpu.VMEM((SUB, sub, half), jnp.bfloat16),
            pltpu.VMEM((2, chunk, half), jnp.float8_e4m3fn),
            pltpu.VMEM((2, chunk, half), jnp.float8_e4m3fn),
            pltpu.VMEM((N_DEV, 8, 128), jnp.float32),
            pltpu.SemaphoreType.DMA((N_DEV - 1, SUB)),
            pltpu.SemaphoreType.DMA((N_DEV - 1, SUB)),
            pltpu.SemaphoreType.DMA((N_DEV - 1, SUB)),
            pltpu.SemaphoreType.DMA((N_DEV - 1, SUB)),
            pltpu.SemaphoreType.DMA((N_DEV - 1, SUB)),
            pltpu.SemaphoreType.DMA((N_DEV - 1, SUB)),
            pltpu.SemaphoreType.DMA((N_DEV - 1, SUB)),
            pltpu.SemaphoreType.DMA((N_DEV - 1, SUB)),
            pltpu.SemaphoreType.DMA((N_DEV - 1,)),
            pltpu.SemaphoreType.DMA((N_DEV - 1,)),
            pltpu.SemaphoreType.REGULAR,
            pltpu.SemaphoreType.REGULAR,
            pltpu.SemaphoreType.DMA((SUB,)),
            pltpu.SemaphoreType.DMA((SUB,)),
            pltpu.SemaphoreType.DMA,
        ],
        compiler_params=pltpu.CompilerParams(
            collective_id=0, vmem_limit_bytes=100 * 1024 * 1024
        ),
    )(x, w)
